# baseline (speedup 1.0000x reference)
import functools

import jax
import jax.numpy as jnp
from jax.experimental import pallas as pl
from jax.experimental.pallas import tpu as pltpu

N_BINS = 11


def _stats(x, labels, lo, hi, n_cols):
    m = jnp.max(x, axis=1, keepdims=True)
    s = jnp.sum(jnp.exp(x - m), axis=1, keepdims=True)
    conf = 1.0 / s
    col = jax.lax.broadcasted_iota(jnp.int32, x.shape, 1)
    xl = jnp.max(jnp.where(col == labels, x, -3.0e38), axis=1, keepdims=True)
    acc = (xl == m).astype(jnp.float32)
    mask = ((conf > lo) & (conf <= hi)).astype(jnp.float32)
    return (jnp.sum(mask, axis=0, keepdims=True),
            jnp.sum(mask * acc, axis=0, keepdims=True),
            jnp.sum(mask * conf, axis=0, keepdims=True))


def _ece_kernel(n_rows, n_cols, nb, xa_ref, xb_ref, labels_ref, bounds_ref,
                ece_ref, accs_ref, confs_ref, cnt_s, asum_s, csum_s):
    i = pl.program_id(0)

    @pl.when(i == 0)
    def _init():
        cnt_s[...] = jnp.zeros_like(cnt_s)
        asum_s[...] = jnp.zeros_like(asum_s)
        csum_s[...] = jnp.zeros_like(csum_s)

    lo = bounds_ref[0:1, 0:N_BINS]
    hi = bounds_ref[0:1, 1:N_BINS + 1]
    half = xa_ref.shape[0]
    labs = labels_ref[0]

    c1, a1, s1 = _stats(xa_ref[...], labs[:half], lo, hi, n_cols)
    c2, a2, s2 = _stats(xb_ref[...], labs[half:], lo, hi, n_cols)
    cnt_s[...] += c1 + c2
    asum_s[...] += a1 + a2
    csum_s[...] += s1 + s2

    @pl.when(i == nb - 1)
    def _fin():
        cnt = cnt_s[...]
        prop = cnt / jnp.float32(n_rows)
        safe = jnp.maximum(cnt, 1.0)
        acc_in = asum_s[...] / safe
        conf_in = csum_s[...] / safe
        nonempty = cnt > 0
        contrib = jnp.where(nonempty, jnp.abs(conf_in - acc_in) * prop, 0.0)
        ece_ref[...] = jnp.sum(contrib, axis=1, keepdims=True)
        accs_ref[...] = jnp.where(nonempty, acc_in, 0.0)
        confs_ref[...] = jnp.where(nonempty, conf_in, 0.0)


def kernel(logits, labels):
    n_rows, n_cols = logits.shape
    block_r = 2048
    nb = n_rows // (2 * block_r)
    labels3 = labels.reshape(nb, 2 * block_r, 1)
    bounds = jnp.linspace(0.0, 1.0, N_BINS + 1).astype(jnp.float32)
    bounds = bounds.reshape(1, N_BINS + 1)

    body = functools.partial(_ece_kernel, n_rows, n_cols, nb)
    ece2, accs2, confs2 = pl.pallas_call(
        body,
        grid=(nb,),
        in_specs=[
            pl.BlockSpec((block_r, n_cols), lambda i: (2 * i, 0)),
            pl.BlockSpec((block_r, n_cols), lambda i: (2 * i + 1, 0)),
            pl.BlockSpec((1, 2 * block_r, 1), lambda i: (i, 0, 0)),
            pl.BlockSpec((1, N_BINS + 1), lambda i: (0, 0)),
        ],
        out_specs=[
            pl.BlockSpec((1, 1), lambda i: (0, 0)),
            pl.BlockSpec((1, N_BINS), lambda i: (0, 0)),
            pl.BlockSpec((1, N_BINS), lambda i: (0, 0)),
        ],
        out_shape=[
            jax.ShapeDtypeStruct((1, 1), jnp.float32),
            jax.ShapeDtypeStruct((1, N_BINS), jnp.float32),
            jax.ShapeDtypeStruct((1, N_BINS), jnp.float32),
        ],
        scratch_shapes=[
            pltpu.VMEM((1, N_BINS), jnp.float32),
            pltpu.VMEM((1, N_BINS), jnp.float32),
            pltpu.VMEM((1, N_BINS), jnp.float32),
        ],
    )(logits, logits, labels3, bounds)
    return (ece2.reshape(1), accs2.reshape(N_BINS), confs2.reshape(N_BINS))
